# baseline (speedup 1.0000x reference)
import functools

import jax
import jax.numpy as jnp
from jax.experimental import pallas as pl
from jax.experimental.pallas import tpu as pltpu


def _split(a):
    bits = jax.lax.bitcast_convert_type(a, jnp.uint32)
    hi_f = jax.lax.bitcast_convert_type(
        (bits + jnp.uint32(0x8000)) & jnp.uint32(0xFFFF0000), jnp.float32)
    return hi_f.astype(jnp.bfloat16), (a - hi_f).astype(jnp.bfloat16)


def _chamfer_body(xt_ref, gt_ref, out_f_ref, out_c_ref,
                  rhs_ref, dbuf_ref, colmin_f_ref, colmin_c_ref, acc_ref,
                  *, nb_fine, nb_total, m, n_blk, n_fine, n_coarse, batches):
    b = pl.program_id(0)
    i = pl.program_id(1)

    @pl.when(jnp.logical_and(b == 0, i == 0))
    def _():
        out_f_ref[0, 0] = 0.0
        out_c_ref[0, 0] = 0.0

    @pl.when(i == 0)
    def _():
        y = gt_ref[0]
        ysq = jnp.sum(y * y, axis=0, keepdims=True)
        y_hi, y_lo = _split(y)
        ysq_hi, ysq_lo = _split(ysq)
        ones_y = jnp.ones((1, m), jnp.bfloat16)
        rhs_ref[...] = jnp.concatenate(
            [y_hi, y_hi, y_lo, ysq_hi, ysq_lo, ones_y, ones_y], axis=0)
        acc_ref[0] = 0.0
        acc_ref[1] = 0.0
        colmin_f_ref[...] = jnp.full((1, m), jnp.inf, jnp.float32)
        colmin_c_ref[...] = jnp.full((1, m), jnp.inf, jnp.float32)

    @pl.when(i < nb_total)
    def _():
        xt = xt_ref[0]
        tx_hi, tx_lo = _split(-2.0 * xt)
        xsq = jnp.sum(xt * xt, axis=0, keepdims=True)
        xsq_hi, xsq_lo = _split(xsq)
        ones_x = jnp.ones((1, n_blk), jnp.bfloat16)
        lhs = jnp.concatenate(
            [tx_hi, tx_lo, tx_hi, ones_x, ones_x, xsq_hi, xsq_lo], axis=0)
        dbuf_ref[pl.ds((i % 2) * n_blk, n_blk), :] = jax.lax.dot_general(
            lhs, rhs_ref[...], (((0,), (0,)), ((), ())),
            preferred_element_type=jnp.float32)

    @pl.when(i > 0)
    def _():
        j = i - 1
        d = dbuf_ref[pl.ds((j % 2) * n_blk, n_blk), :]
        row_sum = jnp.sum(jnp.min(d, axis=1))
        col_min = jnp.min(d, axis=0)[None, :]

        @pl.when(j < nb_fine)
        def _():
            acc_ref[0] += row_sum
            colmin_f_ref[...] = jnp.minimum(colmin_f_ref[...], col_min)

        @pl.when(j >= nb_fine)
        def _():
            acc_ref[1] += row_sum
            colmin_c_ref[...] = jnp.minimum(colmin_c_ref[...], col_min)

    @pl.when(i == nb_total)
    def _():
        loss_f = acc_ref[0] / n_fine + jnp.sum(colmin_f_ref[...]) / m
        loss_c = acc_ref[1] / n_coarse + jnp.sum(colmin_c_ref[...]) / m
        out_f_ref[0, 0] += loss_f / batches
        out_c_ref[0, 0] += loss_c / batches


def kernel(coarse, fine, gt, alpha):
    batches, n_fine, _ = fine.shape
    n_coarse = coarse.shape[1]
    m = gt.shape[2]
    n_blk = n_coarse
    nb_fine = n_fine // n_blk
    nb_total = nb_fine + 1

    xt = jnp.concatenate(
        [jnp.transpose(fine, (0, 2, 1)), jnp.transpose(coarse, (0, 2, 1))],
        axis=2)

    body = functools.partial(
        _chamfer_body, nb_fine=nb_fine, nb_total=nb_total, m=m, n_blk=n_blk,
        n_fine=n_fine, n_coarse=n_coarse, batches=batches)

    out_f, out_c = pl.pallas_call(
        body,
        grid=(batches, nb_total + 1),
        in_specs=[
            pl.BlockSpec((1, 3, n_blk),
                         lambda b, i, _nbt=nb_total: (b, 0, jnp.minimum(i, _nbt - 1))),
            pl.BlockSpec((1, 3, m), lambda b, i: (b, 0, 0)),
        ],
        out_specs=[
            pl.BlockSpec(memory_space=pltpu.SMEM),
            pl.BlockSpec(memory_space=pltpu.SMEM),
        ],
        out_shape=[
            jax.ShapeDtypeStruct((1, 1), jnp.float32),
            jax.ShapeDtypeStruct((1, 1), jnp.float32),
        ],
        scratch_shapes=[
            pltpu.VMEM((13, m), jnp.bfloat16),
            pltpu.VMEM((2 * n_blk, m), jnp.float32),
            pltpu.VMEM((1, m), jnp.float32),
            pltpu.VMEM((1, m), jnp.float32),
            pltpu.SMEM((2,), jnp.float32),
        ],
    )(xt, gt)

    loss_fine = out_f[0, 0]
    loss_coarse = out_c[0, 0]
    a = jnp.reshape(alpha, ())
    loss = loss_coarse + a * loss_fine
    return (loss, loss_coarse, loss_fine)


# speedup vs baseline: 1.3627x; 1.3627x over previous
import functools

import jax
import jax.numpy as jnp
from jax.experimental import pallas as pl
from jax.experimental.pallas import tpu as pltpu


def _split(a):
    bits = jax.lax.bitcast_convert_type(a, jnp.uint32)
    hi_f = jax.lax.bitcast_convert_type(
        (bits + jnp.uint32(0x8000)) & jnp.uint32(0xFFFF0000), jnp.float32)
    return hi_f.astype(jnp.bfloat16), (a - hi_f).astype(jnp.bfloat16)


def _chamfer_body(xt_ref, gt_ref, out_f_ref, out_c_ref,
                  rhs_ref, colmin_f_ref, colmin_c_ref, acc_ref,
                  *, nb_fine, nb_total, m, n_blk, n_fine, n_coarse, batches):
    b = pl.program_id(0)
    i = pl.program_id(1)

    @pl.when(jnp.logical_and(b == 0, i == 0))
    def _():
        out_f_ref[0, 0] = 0.0
        out_c_ref[0, 0] = 0.0

    @pl.when(i == 0)
    def _():
        y = gt_ref[0]
        ysq = jnp.sum(y * y, axis=0, keepdims=True)
        y_hi, y_lo = _split(y)
        ysq_hi, ysq_lo = _split(ysq)
        ones_y = jnp.ones((1, m), jnp.bfloat16)
        rhs_ref[...] = jnp.concatenate(
            [y_hi, y_hi, y_lo, ysq_hi, ysq_lo, ones_y, ones_y], axis=0)
        acc_ref[0] = 0.0
        acc_ref[1] = 0.0
        colmin_f_ref[...] = jnp.full((1, m), jnp.inf, jnp.float32)
        colmin_c_ref[...] = jnp.full((1, m), jnp.inf, jnp.float32)

    xt = xt_ref[0]
    tx_hi, tx_lo = _split(-2.0 * xt)
    xsq = jnp.sum(xt * xt, axis=0, keepdims=True)
    xsq_hi, xsq_lo = _split(xsq)
    ones_x = jnp.ones((1, n_blk), jnp.bfloat16)
    lhs = jnp.concatenate(
        [tx_hi, tx_lo, tx_hi, ones_x, ones_x, xsq_hi, xsq_lo], axis=0)

    d = jax.lax.dot_general(
        lhs, rhs_ref[...], (((0,), (0,)), ((), ())),
        preferred_element_type=jnp.float32)

    row_sum = jnp.sum(jnp.min(d, axis=1))
    col_min = jnp.min(d, axis=0)[None, :]

    @pl.when(i < nb_fine)
    def _():
        acc_ref[0] += row_sum
        colmin_f_ref[...] = jnp.minimum(colmin_f_ref[...], col_min)

    @pl.when(i >= nb_fine)
    def _():
        acc_ref[1] += row_sum
        colmin_c_ref[...] = jnp.minimum(colmin_c_ref[...], col_min)

    @pl.when(i == nb_total - 1)
    def _():
        loss_f = acc_ref[0] / n_fine + jnp.sum(colmin_f_ref[...]) / m
        loss_c = acc_ref[1] / n_coarse + jnp.sum(colmin_c_ref[...]) / m
        out_f_ref[0, 0] += loss_f / batches
        out_c_ref[0, 0] += loss_c / batches


def kernel(coarse, fine, gt, alpha):
    batches, n_fine, _ = fine.shape
    n_coarse = coarse.shape[1]
    m = gt.shape[2]
    n_blk = 512
    nb_fine = n_fine // n_blk
    nb_total = (n_fine + n_coarse) // n_blk

    xt = jnp.concatenate(
        [jnp.transpose(fine, (0, 2, 1)), jnp.transpose(coarse, (0, 2, 1))],
        axis=2)

    body = functools.partial(
        _chamfer_body, nb_fine=nb_fine, nb_total=nb_total, m=m, n_blk=n_blk,
        n_fine=n_fine, n_coarse=n_coarse, batches=batches)

    out_f, out_c = pl.pallas_call(
        body,
        grid=(batches, nb_total),
        in_specs=[
            pl.BlockSpec((1, 3, n_blk), lambda b, i: (b, 0, i)),
            pl.BlockSpec((1, 3, m), lambda b, i: (b, 0, 0)),
        ],
        out_specs=[
            pl.BlockSpec(memory_space=pltpu.SMEM),
            pl.BlockSpec(memory_space=pltpu.SMEM),
        ],
        out_shape=[
            jax.ShapeDtypeStruct((1, 1), jnp.float32),
            jax.ShapeDtypeStruct((1, 1), jnp.float32),
        ],
        scratch_shapes=[
            pltpu.VMEM((13, m), jnp.bfloat16),
            pltpu.VMEM((1, m), jnp.float32),
            pltpu.VMEM((1, m), jnp.float32),
            pltpu.SMEM((2,), jnp.float32),
        ],
    )(xt, gt)

    loss_fine = out_f[0, 0]
    loss_coarse = out_c[0, 0]
    a = jnp.reshape(alpha, ())
    loss = loss_coarse + a * loss_fine
    return (loss, loss_coarse, loss_fine)


# speedup vs baseline: 1.5691x; 1.1515x over previous
import functools

import jax
import jax.numpy as jnp
from jax.experimental import pallas as pl
from jax.experimental.pallas import tpu as pltpu


def _split(a):
    bits = jax.lax.bitcast_convert_type(a, jnp.uint32)
    hi_f = jax.lax.bitcast_convert_type(
        (bits + jnp.uint32(0x8000)) & jnp.uint32(0xFFFF0000), jnp.float32)
    return hi_f.astype(jnp.bfloat16), (a - hi_f).astype(jnp.bfloat16)


def _chamfer_body(xt_ref, gt_ref, out_f_ref, out_c_ref,
                  rhs_ref, colmin_f_ref, colmin_c_ref, acc_ref,
                  *, nb_fine, nb_total, m, n_blk, n_fine, n_coarse, batches):
    b = pl.program_id(0)
    i = pl.program_id(1)

    @pl.when(jnp.logical_and(b == 0, i == 0))
    def _():
        out_f_ref[0, 0] = 0.0
        out_c_ref[0, 0] = 0.0

    @pl.when(i == 0)
    def _():
        y = gt_ref[0]
        ysq = jnp.sum(y * y, axis=0, keepdims=True)
        y_hi, y_lo = _split(y)
        ysq_hi, ysq_lo = _split(ysq)
        ones_y = jnp.ones((1, m), jnp.bfloat16)
        rhs_ref[...] = jnp.concatenate(
            [y_hi, y_hi, y_lo, ysq_hi, ysq_lo, ones_y, ones_y], axis=0)
        acc_ref[0] = 0.0
        acc_ref[1] = 0.0
        colmin_f_ref[...] = jnp.full((1, m), jnp.inf, jnp.float32)
        colmin_c_ref[...] = jnp.full((1, m), jnp.inf, jnp.float32)

    xt = xt_ref[0]
    tx_hi, tx_lo = _split(-2.0 * xt)
    xsq = jnp.sum(xt * xt, axis=0, keepdims=True)
    xsq_hi, xsq_lo = _split(xsq)
    ones_x = jnp.ones((1, n_blk), jnp.bfloat16)
    lhs = jnp.concatenate(
        [tx_hi, tx_lo, tx_hi, ones_x, ones_x, xsq_hi, xsq_lo], axis=0)

    d = jax.lax.dot_general(
        lhs, rhs_ref[...], (((0,), (0,)), ((), ())),
        preferred_element_type=jnp.float32)

    row_sum = jnp.sum(jnp.min(d, axis=1))
    col_min = jnp.min(d, axis=0)[None, :]

    @pl.when(i < nb_fine)
    def _():
        acc_ref[0] += row_sum
        colmin_f_ref[...] = jnp.minimum(colmin_f_ref[...], col_min)

    @pl.when(i >= nb_fine)
    def _():
        acc_ref[1] += row_sum
        colmin_c_ref[...] = jnp.minimum(colmin_c_ref[...], col_min)

    @pl.when(i == nb_total - 1)
    def _():
        loss_f = acc_ref[0] / n_fine + jnp.sum(colmin_f_ref[...]) / m
        loss_c = acc_ref[1] / n_coarse + jnp.sum(colmin_c_ref[...]) / m
        out_f_ref[0, 0] += loss_f / batches
        out_c_ref[0, 0] += loss_c / batches


def kernel(coarse, fine, gt, alpha):
    batches, n_fine, _ = fine.shape
    n_coarse = coarse.shape[1]
    m = gt.shape[2]
    n_blk = n_coarse
    nb_fine = n_fine // n_blk
    nb_total = nb_fine + 1

    xt = jnp.concatenate(
        [jnp.transpose(fine, (0, 2, 1)), jnp.transpose(coarse, (0, 2, 1))],
        axis=2)

    body = functools.partial(
        _chamfer_body, nb_fine=nb_fine, nb_total=nb_total, m=m, n_blk=n_blk,
        n_fine=n_fine, n_coarse=n_coarse, batches=batches)

    out_f, out_c = pl.pallas_call(
        body,
        grid=(batches, nb_total),
        in_specs=[
            pl.BlockSpec((1, 3, n_blk), lambda b, i: (b, 0, i)),
            pl.BlockSpec((1, 3, m), lambda b, i: (b, 0, 0)),
        ],
        out_specs=[
            pl.BlockSpec(memory_space=pltpu.SMEM),
            pl.BlockSpec(memory_space=pltpu.SMEM),
        ],
        out_shape=[
            jax.ShapeDtypeStruct((1, 1), jnp.float32),
            jax.ShapeDtypeStruct((1, 1), jnp.float32),
        ],
        scratch_shapes=[
            pltpu.VMEM((13, m), jnp.bfloat16),
            pltpu.VMEM((1, m), jnp.float32),
            pltpu.VMEM((1, m), jnp.float32),
            pltpu.SMEM((2,), jnp.float32),
        ],
    )(xt, gt)

    loss_fine = out_f[0, 0]
    loss_coarse = out_c[0, 0]
    a = jnp.reshape(alpha, ())
    loss = loss_coarse + a * loss_fine
    return (loss, loss_coarse, loss_fine)
